# Initial kernel scaffold; baseline (speedup 1.0000x reference)
import jax
import jax.numpy as jnp
from jax.experimental import pallas as pl


def kernel(x_src, x_dst, edge_index, edge_attr, Wl_f, Wr_f, We_f, att_f, b_f, Wl_b, Wr_b, We_b, att_b, b_b):
    raise NotImplementedError("write your pallas kernel here")


# speedup vs baseline: 7.6398x; 7.6398x over previous
import functools

import jax
import jax.numpy as jnp
from jax import lax
from jax.experimental import pallas as pl
from jax.experimental.pallas import tpu as pltpu
from jax.experimental.pallas import tpu_sc as plsc

NEG_SLOPE = 0.2
NC = 2
NS = 16
NW = NC * NS
LANES = 16


def _tc_pre(x_src, x_dst, Wl_b, Wr_b, Wl_f, Wr_f):
    n, d = x_src.shape
    out = Wl_b.shape[1]

    def body(xs, xd, wlb, wrb, wlf, wrf, hlb, hrb, hlf, hrf):
        hlb[...] = jnp.dot(xs[...], wlb[...], preferred_element_type=jnp.float32)
        hrb[...] = jnp.dot(xd[...], wrb[...], preferred_element_type=jnp.float32)
        hlf[...] = jnp.dot(xd[...], wlf[...], preferred_element_type=jnp.float32)
        hrf[...] = jnp.dot(xs[...], wrf[...], preferred_element_type=jnp.float32)

    o = jax.ShapeDtypeStruct((n, out), jnp.float32)
    return pl.pallas_call(body, out_shape=[o, o, o, o])(
        x_src, x_dst, Wl_b, Wr_b, Wl_f, Wr_f)


def _tc_edge(ea, We_b, We_f):
    e, de = ea.shape
    out = We_b.shape[1]
    be = 8000
    grid = e // be

    def body(ea_ref, wb_ref, wf_ref, ob_ref, of_ref):
        ob_ref[...] = jnp.dot(ea_ref[...], wb_ref[...], preferred_element_type=jnp.float32)
        of_ref[...] = jnp.dot(ea_ref[...], wf_ref[...], preferred_element_type=jnp.float32)

    o = jax.ShapeDtypeStruct((e, out), jnp.float32)
    return pl.pallas_call(
        body,
        grid=(grid,),
        in_specs=[
            pl.BlockSpec((be, de), lambda i: (i, 0)),
            pl.BlockSpec((de, out), lambda i: (0, 0)),
            pl.BlockSpec((de, out), lambda i: (0, 0)),
        ],
        out_specs=[pl.BlockSpec((be, out), lambda i: (i, 0))] * 2,
        out_shape=[o, o],
    )(ea, We_b, We_f)


def _tc_fin(num, den, b):
    n, out = den.shape[1], num.shape[2]

    def body(num_ref, den_ref, b_ref, o_ref):
        s = num_ref[0, :n, :] + num_ref[1, :n, :]
        d = jnp.sum(den_ref[...], axis=0) + 1e-16
        o_ref[...] = s / d[:, None] + b_ref[...]

    return pl.pallas_call(
        body, out_shape=jax.ShapeDtypeStruct((n, out), jnp.float32))(num, den, b)


def _sc_pass(hl, hr, he, src, dst, att):
    n, out = hl.shape
    e = src.shape[0]
    epw = e // NW
    c = 80
    nchunk = epw // c
    ngrp = c // LANES
    nj = out // LANES
    n_pad = ((n + NS * 128 - 1) // (NS * 128)) * (NS * 128)
    rows_per_tile = n_pad // NS
    zrows = c
    assert epw * NW == e and nchunk * c == epw
    assert (rows_per_tile // zrows) * zrows == rows_per_tile

    mesh = plsc.VectorSubcoreMesh(core_axis_name="c", subcore_axis_name="s")

    @functools.partial(
        pl.kernel,
        out_type=[jax.ShapeDtypeStruct((NC, n_pad, out), jnp.float32),
                  jax.ShapeDtypeStruct((NW, n), jnp.float32)],
        mesh=mesh,
        compiler_params=pltpu.CompilerParams(needs_layout_passes=False),
        scratch_types=[
            pltpu.VMEM((c,), jnp.int32),
            pltpu.VMEM((c,), jnp.int32),
            pltpu.VMEM((c, out), jnp.float32),
            pltpu.VMEM((c, out), jnp.float32),
            pltpu.VMEM((c, out), jnp.float32),
            pltpu.VMEM((LANES * LANES,), jnp.float32),
            pltpu.VMEM((out,), jnp.float32),
            pltpu.VMEM((n,), jnp.float32),
            pltpu.VMEM_SHARED((n_pad, out), jnp.float32),
            pltpu.SemaphoreType.DMA,
            pltpu.SemaphoreType.DMA,
        ],
    )
    def k(hl_h, hr_h, he_h, src_h, dst_h, att_h, num_h, den_h,
          idx_s, idx_d, gs, gd, hev, accb, attv, dloc, nums,
          sem1, sem2):
        cid = lax.axis_index("c")
        sid = lax.axis_index("s")
        wid = cid * NS + sid
        z16 = jnp.zeros((LANES,), jnp.float32)

        def zero_gs(i, carry):
            for j in range(nj):
                gs[i, pl.ds(j * LANES, LANES)] = z16
            return carry
        lax.fori_loop(0, zrows, zero_gs, 0)

        def zero_dloc(i, carry):
            dloc[pl.ds(pl.multiple_of(i * LANES, LANES), LANES)] = z16
            return carry
        lax.fori_loop(0, n // LANES, zero_dloc, 0)

        for kb in range(rows_per_tile // zrows):
            rb = sid * rows_per_tile + kb * zrows
            pltpu.sync_copy(gs, nums.at[pl.ds(rb, zrows)])
        pltpu.sync_copy(att_h, attv)
        plsc.subcore_barrier()

        def chunk_body(ci, carry):
            base = pl.multiple_of(wid * epw + ci * c, 8)
            pltpu.sync_copy(src_h.at[pl.ds(base, c)], idx_s)
            pltpu.sync_copy(dst_h.at[pl.ds(base, c)], idx_d)
            cp1 = pltpu.async_copy(hl_h.at[idx_s], gs, sem1)
            cp2 = pltpu.async_copy(hr_h.at[idx_d], gd, sem2)
            pltpu.sync_copy(he_h.at[pl.ds(base, c)], hev)
            cp1.wait()
            cp2.wait()

            def group_body(gi, gcarry):
                goff = gi * LANES
                for i in range(LANES):
                    r = goff + i
                    acc = None
                    for j in range(nj):
                        sl = pl.ds(j * LANES, LANES)
                        m = gs[r, sl] + gd[r, sl] + hev[r, sl]
                        m = jnp.maximum(m, NEG_SLOPE * m)
                        t = m * attv[sl]
                        acc = t if acc is None else acc + t
                    accb[pl.ds(i * LANES, LANES)] = acc
                rowb = lax.iota(jnp.int32, LANES) * LANES
                al = None
                for j in range(LANES):
                    t = plsc.load_gather(accb, [rowb + j])
                    al = t if al is None else al + t
                ex = jnp.exp(al)
                dvec = idx_d[pl.ds(goff, LANES)]
                plsc.addupdate_scatter(dloc, [dvec], ex)
                for i in range(LANES):
                    r = goff + i
                    ei = ex[i]
                    for j in range(nj):
                        sl = pl.ds(j * LANES, LANES)
                        gs[r, sl] = gs[r, sl] * ei
                return gcarry
            lax.fori_loop(0, ngrp, group_body, 0)
            pltpu.sync_copy(gs, nums.at[idx_d], add=True)
            return carry
        lax.fori_loop(0, nchunk, chunk_body, 0)

        plsc.subcore_barrier()
        for kb in range(rows_per_tile // zrows):
            rb = sid * rows_per_tile + kb * zrows
            pltpu.sync_copy(nums.at[pl.ds(rb, zrows)],
                            num_h.at[cid, pl.ds(rb, zrows)])
        pltpu.sync_copy(dloc, den_h.at[wid])

    return k(hl, hr, he, src, dst, att)


def kernel(x_src, x_dst, edge_index, edge_attr,
           Wl_f, Wr_f, We_f, att_f, b_f,
           Wl_b, Wr_b, We_b, att_b, b_b):
    src = edge_index[0]
    dst = edge_index[1]
    hlb, hrb, hlf, hrf = _tc_pre(x_src, x_dst, Wl_b, Wr_b, Wl_f, Wr_f)
    heb, hef = _tc_edge(edge_attr, We_b, We_f)
    numb, denb = _sc_pass(hlb, hrb, heb, src, dst, att_b)
    numf, denf = _sc_pass(hlf, hrf, hef, dst, src, att_f)
    out_bwd = _tc_fin(numb, denb, b_b)
    out_fwd = _tc_fin(numf, denf, b_f)
    return (out_bwd, out_fwd)
